# Initial kernel scaffold; baseline (speedup 1.0000x reference)
import jax
import jax.numpy as jnp
from jax.experimental import pallas as pl


def kernel(input, edge_index, edge_weight, weight, bias):
    raise NotImplementedError("write your pallas kernel here")


# speedup vs baseline: 5.5124x; 5.5124x over previous
import functools

import jax
import jax.numpy as jnp
from jax import lax
from jax.experimental import pallas as pl
from jax.experimental.pallas import tpu as pltpu
from jax.experimental.pallas import tpu_sc as plsc

N = 10000
E = 320000
D = 128

NC = 2
NS = 16
LANES = 16

CHUNK = 80
EDGES_PER_CORE = E // NC
EDGES_PER_TILE = EDGES_PER_CORE // NS
SUB = 2000
NBLK = EDGES_PER_TILE // SUB
CHUNKS_PER_BLK = SUB // CHUNK
ROWS_PER_TILE = 640
ACC_ROWS = NS * ROWS_PER_TILE
INIT_ROWS = 80

_BCAST_DNUMS = lax.GatherDimensionNumbers(
    offset_dims=(), collapsed_slice_dims=(0,), start_index_map=(0,))


def _matmul_body(x_ref, w_ref, o_ref):
    o_ref[...] = jnp.dot(x_ref[...], w_ref[...],
                         preferred_element_type=jnp.float32)


def _matmul(x, w):
    blk = 1000
    return pl.pallas_call(
        _matmul_body,
        grid=(N // blk,),
        in_specs=[
            pl.BlockSpec((blk, D), lambda i: (i, 0)),
            pl.BlockSpec((D, D), lambda i: (0, 0)),
        ],
        out_specs=pl.BlockSpec((blk, D), lambda i: (i, 0)),
        out_shape=jax.ShapeDtypeStruct((N, D), jnp.float32),
    )(x, w)


def _sc_body(support_hbm, src_hbm, dst_hbm, ew_hbm, init_hbm, out_hbm,
             srcs_v, dsts_v, ws_v, sidx_v, didx_v, rows_v, init_v, acc, sem):
    c = lax.axis_index("c")
    s = lax.axis_index("s")

    pltpu.sync_copy(init_hbm.at[pl.ds(c * INIT_ROWS, INIT_ROWS)], init_v)
    row0 = s * ROWS_PER_TILE
    for k in range(ROWS_PER_TILE // INIT_ROWS):
        pltpu.sync_copy(init_v, acc.at[pl.ds(row0 + k * INIT_ROWS, INIT_ROWS)])
    plsc.subcore_barrier()

    ebase = c * EDGES_PER_CORE + s * EDGES_PER_TILE

    def blk_body(b, carry0):
        bbase = ebase + b * SUB
        pltpu.sync_copy(src_hbm.at[pl.ds(bbase, SUB)], srcs_v)
        pltpu.sync_copy(dst_hbm.at[pl.ds(bbase, SUB)], dsts_v)
        pltpu.sync_copy(ew_hbm.at[pl.ds(bbase, SUB)], ws_v)
        lax.fori_loop(0, CHUNKS_PER_BLK, chunk_body, 0)
        return carry0

    def chunk_body(k, carry):
        base = k * CHUNK
        for t in range(CHUNK // LANES):
            sl = pl.ds(t * LANES, LANES)
            sidx_v[sl] = srcs_v[pl.ds(base + t * LANES, LANES)]
            didx_v[sl] = dsts_v[pl.ds(base + t * LANES, LANES)]
        pltpu.async_copy(support_hbm.at[sidx_v], rows_v, sem).wait()

        def group_body(g, carry2):
            wv = ws_v[pl.ds(base + g * LANES, LANES)]

            def row_body(r, carry3):
                wb = lax.gather(
                    wv, jnp.full((LANES, 1), r, jnp.int32),
                    _BCAST_DNUMS, slice_sizes=(1,),
                    mode=lax.GatherScatterMode.PROMISE_IN_BOUNDS)
                row = g * LANES + r
                for j in range(D // LANES):
                    sl2 = (row, pl.ds(j * LANES, LANES))
                    rows_v[sl2] = rows_v[sl2] * wb
                return carry3

            return lax.fori_loop(0, LANES, row_body, carry2)

        lax.fori_loop(0, CHUNK // LANES, group_body, 0)

        pltpu.sync_copy(rows_v, acc.at[didx_v], add=True)
        return carry

    lax.fori_loop(0, NBLK, blk_body, 0)

    plsc.subcore_barrier()
    obase = c * ACC_ROWS + s * ROWS_PER_TILE
    for k in range(ROWS_PER_TILE // INIT_ROWS):
        pltpu.sync_copy(acc.at[pl.ds(row0 + k * INIT_ROWS, INIT_ROWS)],
                        out_hbm.at[pl.ds(obase + k * INIT_ROWS, INIT_ROWS)])


def _sc_aggregate(support, src, dst, ew, init):
    mesh = plsc.VectorSubcoreMesh(core_axis_name="c", subcore_axis_name="s")
    f = pl.kernel(
        _sc_body,
        out_type=jax.ShapeDtypeStruct((NC * ACC_ROWS, D), jnp.float32),
        mesh=mesh,
        scratch_types=[
            pltpu.VMEM((SUB,), jnp.int32),
            pltpu.VMEM((SUB,), jnp.int32),
            pltpu.VMEM((SUB,), jnp.float32),
            pltpu.VMEM((CHUNK,), jnp.int32),
            pltpu.VMEM((CHUNK,), jnp.int32),
            pltpu.VMEM((CHUNK, D), jnp.float32),
            pltpu.VMEM((INIT_ROWS, D), jnp.float32),
            pltpu.VMEM_SHARED((ACC_ROWS, D), jnp.float32),
            pltpu.SemaphoreType.DMA,
        ],
    )
    return f(support, src, dst, ew, init)


def _combine_body(a_ref, b_ref, o_ref):
    o_ref[...] = a_ref[...] + b_ref[...]


def _combine(part):
    blk = 80
    return pl.pallas_call(
        _combine_body,
        grid=(N // blk,),
        in_specs=[
            pl.BlockSpec((blk, D), lambda i: (i, 0)),
            pl.BlockSpec((blk, D), lambda i: (i + ACC_ROWS // blk, 0)),
        ],
        out_specs=pl.BlockSpec((blk, D), lambda i: (i, 0)),
        out_shape=jax.ShapeDtypeStruct((N, D), jnp.float32),
    )(part, part)


@jax.jit
def _run(input, edge_index, edge_weight, weight, bias):
    support = _matmul(input, weight)
    src = edge_index[0]
    dst = edge_index[1]
    init = jnp.concatenate(
        [jnp.tile(bias[None, :], (INIT_ROWS, 1)),
         jnp.zeros((INIT_ROWS, D), jnp.float32)], axis=0)
    part = _sc_aggregate(support, src, dst, edge_weight, init)
    return _combine(part)


def kernel(input, edge_index, edge_weight, weight, bias):
    return _run(input, edge_index, edge_weight, weight, bias)
